# baseline (speedup 1.0000x reference)
import functools

import jax
import jax.numpy as jnp
from jax.experimental import pallas as pl
from jax.experimental.pallas import tpu as pltpu

EPS = 1e-5

_OFF_B1, _OFF_G1, _OFF_BE1 = 0, 384, 768
_OFF_B2A, _OFF_B2B = 1152, 1280
_OFF_G2A, _OFF_G2B = 1344, 1472
_OFF_BE2A, _OFF_BE2B = 1536, 1664
_OFF_W3A, _OFF_W3B, _OFF_B3 = 1728, 1856, 1920
_VEC_LEN = 1928


def _lnt(h, g, be):
    mu = jnp.mean(h, axis=0, keepdims=True)
    var = jnp.mean(h * h, axis=0, keepdims=True) - mu * mu
    return jnp.maximum((h - mu) * jax.lax.rsqrt(var + EPS) * g + be, 0.0)


def _bf(a):
    return a.astype(jnp.bfloat16)


def _body(x_ref, lab_ref, w1w_ref, w1c_ref, w1s_ref, w1t_ref,
          w2wc_ref, w2st_ref, vec_ref, o_ref):
    vec = lambda off, ln: vec_ref[pl.ds(off, ln), :]
    xb = _bf(x_ref[...])

    tdot = functools.partial(jax.lax.dot_general,
                             dimension_numbers=(((0,), (1,)), ((), ())),
                             preferred_element_type=jnp.float32)
    b1 = vec(_OFF_B1, 384)
    g1 = vec(_OFF_G1, 384)
    be1 = vec(_OFF_BE1, 384)
    h0 = tdot(_bf(w1w_ref[...]), xb) + b1[0:128]
    h1 = tdot(_bf(w1c_ref[...]), xb) + b1[128:256]
    hs = tdot(_bf(w1s_ref[...]), xb) + b1[256:320]
    htt = tdot(_bf(w1t_ref[...]), xb) + b1[320:384]
    ln0 = _lnt(h0, g1[0:128], be1[0:128])
    ln1 = _lnt(h1, g1[128:256], be1[128:256])
    lns = _lnt(hs, g1[256:320], be1[256:320])
    lnt_ = _lnt(htt, g1[320:384], be1[320:384])

    w2wc = w2wc_ref[...]
    wlanes = jax.lax.broadcasted_iota(jnp.int32, w2wc.shape, 1)
    w2a = jnp.concatenate([jnp.where(wlanes < 64, w2wc, 0.0),
                           jnp.where(wlanes >= 64, w2wc, 0.0)], axis=0)
    ln01 = jnp.concatenate([ln0, ln1], axis=0)
    hat = jax.lax.dot_general(
        _bf(w2a), _bf(ln01), (((0,), (0,)), ((), ())),
        preferred_element_type=jnp.float32)
    hat = hat + vec(_OFF_B2A, 128)

    w2st = w2st_ref[...]
    slanes = jax.lax.broadcasted_iota(jnp.int32, w2st.shape, 1)
    w2b = jnp.concatenate([jnp.where(slanes < 32, w2st, 0.0),
                           jnp.where((slanes >= 32) & (slanes < 64), w2st, 0.0)],
                          axis=0)
    ln23 = jnp.concatenate([lns, lnt_], axis=0)
    hbt = jax.lax.dot_general(
        _bf(w2b), _bf(ln23), (((0,), (0,)), ((), ())),
        preferred_element_type=jnp.float32)
    hbt = hbt[0:64] + vec(_OFF_B2B, 64)

    g2a, be2a = vec(_OFF_G2A, 128), vec(_OFF_BE2A, 128)
    g2b, be2b = vec(_OFF_G2B, 64), vec(_OFF_BE2B, 64)
    lnw2 = _lnt(hat[0:64], g2a[0:64], be2a[0:64])
    lnc2 = _lnt(hat[64:128], g2a[64:128], be2a[64:128])
    lns2 = _lnt(hbt[0:32], g2b[0:32], be2b[0:32])
    lnt2 = _lnt(hbt[32:64], g2b[32:64], be2b[32:64])

    w3a = vec(_OFF_W3A, 128)
    w3b = vec(_OFF_W3B, 64)
    p_w = jnp.sum(lnw2 * w3a[0:64], axis=0, keepdims=True)
    p_c = jnp.sum(lnc2 * w3a[64:128], axis=0, keepdims=True)
    p_s = jnp.sum(lns2 * w3b[0:32], axis=0, keepdims=True)
    p_t = jnp.sum(lnt2 * w3b[32:64], axis=0, keepdims=True)

    lab = lab_ref[...]
    preds = jnp.where(
        lab < 2,
        jnp.where(lab == 0, p_s + vec_ref[_OFF_B3, 0],
                  p_t + vec_ref[_OFF_B3 + 1, 0]),
        jnp.where(lab == 2, p_w + vec_ref[_OFF_B3 + 2, 0],
                  p_c + vec_ref[_OFF_B3 + 3, 0]))
    o_ref[...] = preds


def kernel(x, group_labels, params):
    n, d = x.shape
    blk = 2048
    labels = group_labels.astype(jnp.int32).reshape(1, n)
    pw, pc, ps, pt = (params[k] for k in ("women", "children", "sc", "st"))

    w2wc = jnp.concatenate([pw["W2"], pc["W2"]], axis=1)
    z = jnp.zeros((64, 64), jnp.float32)
    w2st = jnp.concatenate([ps["W2"], pt["W2"], z], axis=1)
    col = lambda a: a[:, None]
    vecs = jnp.concatenate([
        col(pw["b1"]), col(pc["b1"]), col(ps["b1"]), col(pt["b1"]),
        col(pw["g1"]), col(pc["g1"]), col(ps["g1"]), col(pt["g1"]),
        col(pw["be1"]), col(pc["be1"]), col(ps["be1"]), col(pt["be1"]),
        col(pw["b2"]), col(pc["b2"]), col(ps["b2"]), col(pt["b2"]),
        col(pw["g2"]), col(pc["g2"]), col(ps["g2"]), col(pt["g2"]),
        col(pw["be2"]), col(pc["be2"]), col(ps["be2"]), col(pt["be2"]),
        pw["W3"], pc["W3"], ps["W3"], pt["W3"],
        col(ps["b3"]), col(pt["b3"]), col(pw["b3"]), col(pc["b3"]),
        jnp.zeros((4, 1), jnp.float32),
    ], axis=0)
    assert vecs.shape == (_VEC_LEN, 1)

    w1s = [pw["W1"], pc["W1"], ps["W1"], pt["W1"]]
    out = pl.pallas_call(
        _body,
        grid=(n // blk,),
        in_specs=[
            pl.BlockSpec((blk, d), lambda i: (i, 0)),
            pl.BlockSpec((1, blk), lambda i: (0, i)),
        ] + [pl.BlockSpec(w.shape, lambda i: (0, 0)) for w in w1s] + [
            pl.BlockSpec(w2wc.shape, lambda i: (0, 0)),
            pl.BlockSpec(w2st.shape, lambda i: (0, 0)),
            pl.BlockSpec(vecs.shape, lambda i: (0, 0)),
        ],
        out_specs=pl.BlockSpec((1, blk), lambda i: (0, i)),
        out_shape=jax.ShapeDtypeStruct((1, n), x.dtype),
        compiler_params=pltpu.CompilerParams(
            dimension_semantics=("parallel",)),
    )(x, labels, *w1s, w2wc, w2st, vecs)
    return out.reshape(n, 1)


# speedup vs baseline: 1.5778x; 1.5778x over previous
import functools

import jax
import jax.numpy as jnp
from jax.experimental import pallas as pl
from jax.experimental.pallas import tpu as pltpu

EPS = 1e-5

_OFF_B1, _OFF_G1, _OFF_BE1 = 0, 384, 768
_OFF_B2A, _OFF_B2B = 1152, 1280
_OFF_G2A, _OFF_G2B = 1344, 1472
_OFF_BE2A, _OFF_BE2B = 1536, 1664
_OFF_W3A, _OFF_W3B, _OFF_B3 = 1728, 1856, 1920
_VEC_LEN = 1928


def _lnt(h, g, be):
    mu = jnp.mean(h, axis=0, keepdims=True)
    var = jnp.mean(h * h, axis=0, keepdims=True) - mu * mu
    return jnp.maximum((h - mu) * jax.lax.rsqrt(var + EPS) * g + be, 0.0)


def _bf(a):
    return a.astype(jnp.bfloat16)


def _body(x_ref, lab_ref, w1_ref, w2wc_ref, w2st_ref, vec_ref, o_ref):
    vec = lambda off, ln: vec_ref[pl.ds(off, ln), :]
    xb = _bf(x_ref[...])

    ht = jax.lax.dot_general(
        _bf(w1_ref[...]), xb, (((0,), (1,)), ((), ())),
        preferred_element_type=jnp.float32)
    ht = ht + vec(_OFF_B1, 384)
    g1 = vec(_OFF_G1, 384)
    be1 = vec(_OFF_BE1, 384)
    ln0 = _lnt(ht[0:128], g1[0:128], be1[0:128])
    ln1 = _lnt(ht[128:256], g1[128:256], be1[128:256])
    lns = _lnt(ht[256:320], g1[256:320], be1[256:320])
    lnt_ = _lnt(ht[320:384], g1[320:384], be1[320:384])

    w2wc = w2wc_ref[...]
    wlanes = jax.lax.broadcasted_iota(jnp.int32, w2wc.shape, 1)
    w2a = jnp.concatenate([jnp.where(wlanes < 64, w2wc, 0.0),
                           jnp.where(wlanes >= 64, w2wc, 0.0)], axis=0)
    ln01 = jnp.concatenate([ln0, ln1], axis=0)
    hat = jax.lax.dot_general(
        _bf(w2a), _bf(ln01), (((0,), (0,)), ((), ())),
        preferred_element_type=jnp.float32)
    hat = hat + vec(_OFF_B2A, 128)

    w2st = w2st_ref[...]
    slanes = jax.lax.broadcasted_iota(jnp.int32, w2st.shape, 1)
    w2b = jnp.concatenate([jnp.where(slanes < 32, w2st, 0.0),
                           jnp.where((slanes >= 32) & (slanes < 64), w2st, 0.0)],
                          axis=0)
    ln23 = jnp.concatenate([lns, lnt_], axis=0)
    hbt = jax.lax.dot_general(
        _bf(w2b), _bf(ln23), (((0,), (0,)), ((), ())),
        preferred_element_type=jnp.float32)
    hbt = hbt[0:64] + vec(_OFF_B2B, 64)

    g2a, be2a = vec(_OFF_G2A, 128), vec(_OFF_BE2A, 128)
    g2b, be2b = vec(_OFF_G2B, 64), vec(_OFF_BE2B, 64)
    lnw2 = _lnt(hat[0:64], g2a[0:64], be2a[0:64])
    lnc2 = _lnt(hat[64:128], g2a[64:128], be2a[64:128])
    lns2 = _lnt(hbt[0:32], g2b[0:32], be2b[0:32])
    lnt2 = _lnt(hbt[32:64], g2b[32:64], be2b[32:64])

    w3a = vec(_OFF_W3A, 128)
    w3b = vec(_OFF_W3B, 64)
    p_w = jnp.sum(lnw2 * w3a[0:64], axis=0, keepdims=True)
    p_c = jnp.sum(lnc2 * w3a[64:128], axis=0, keepdims=True)
    p_s = jnp.sum(lns2 * w3b[0:32], axis=0, keepdims=True)
    p_t = jnp.sum(lnt2 * w3b[32:64], axis=0, keepdims=True)

    lab = lab_ref[...]
    preds = jnp.where(
        lab < 2,
        jnp.where(lab == 0, p_s + vec_ref[_OFF_B3, 0],
                  p_t + vec_ref[_OFF_B3 + 1, 0]),
        jnp.where(lab == 2, p_w + vec_ref[_OFF_B3 + 2, 0],
                  p_c + vec_ref[_OFF_B3 + 3, 0]))
    o_ref[...] = preds


def kernel(x, group_labels, params):
    n, d = x.shape
    blk = 2048
    labels = group_labels.astype(jnp.int32).reshape(1, n)
    pw, pc, ps, pt = (params[k] for k in ("women", "children", "sc", "st"))

    w1 = jnp.concatenate([pw["W1"], pc["W1"], ps["W1"], pt["W1"]], axis=1)
    w2wc = jnp.concatenate([pw["W2"], pc["W2"]], axis=1)
    z = jnp.zeros((64, 64), jnp.float32)
    w2st = jnp.concatenate([ps["W2"], pt["W2"], z], axis=1)
    col = lambda a: a[:, None]
    vecs = jnp.concatenate([
        col(pw["b1"]), col(pc["b1"]), col(ps["b1"]), col(pt["b1"]),
        col(pw["g1"]), col(pc["g1"]), col(ps["g1"]), col(pt["g1"]),
        col(pw["be1"]), col(pc["be1"]), col(ps["be1"]), col(pt["be1"]),
        col(pw["b2"]), col(pc["b2"]), col(ps["b2"]), col(pt["b2"]),
        col(pw["g2"]), col(pc["g2"]), col(ps["g2"]), col(pt["g2"]),
        col(pw["be2"]), col(pc["be2"]), col(ps["be2"]), col(pt["be2"]),
        pw["W3"], pc["W3"], ps["W3"], pt["W3"],
        col(ps["b3"]), col(pt["b3"]), col(pw["b3"]), col(pc["b3"]),
        jnp.zeros((4, 1), jnp.float32),
    ], axis=0)
    assert vecs.shape == (_VEC_LEN, 1)

    out = pl.pallas_call(
        _body,
        grid=(n // blk,),
        in_specs=[
            pl.BlockSpec((blk, d), lambda i: (i, 0)),
            pl.BlockSpec((1, blk), lambda i: (0, i)),
            pl.BlockSpec(w1.shape, lambda i: (0, 0)),
            pl.BlockSpec(w2wc.shape, lambda i: (0, 0)),
            pl.BlockSpec(w2st.shape, lambda i: (0, 0)),
            pl.BlockSpec(vecs.shape, lambda i: (0, 0)),
        ],
        out_specs=pl.BlockSpec((1, blk), lambda i: (0, i)),
        out_shape=jax.ShapeDtypeStruct((1, n), x.dtype),
        compiler_params=pltpu.CompilerParams(
            dimension_semantics=("arbitrary",)),
    )(x, labels, w1, w2wc, w2st, vecs)
    return out.reshape(n, 1)
